# Initial kernel scaffold; baseline (speedup 1.0000x reference)
import jax
import jax.numpy as jnp
from jax.experimental import pallas as pl


def kernel(ndata, edge_weight, W1, b1, W2, b2, eps1, eps2, eps3, edge_index):
    raise NotImplementedError("write your pallas kernel here")


# speedup vs baseline: 3.7010x; 3.7010x over previous
import dataclasses
import functools

import jax
import jax.numpy as jnp
from jax import lax
from jax.experimental import pallas as pl
from jax.experimental.pallas import tpu as pltpu
from jax.experimental.pallas import tpu_sc as plsc

_N = 10000
_D = 128
_NP = 10240
_K = 80
_NC = 2
_NS = 16
_NW = _NC * _NS
_L = 16


def _mm_body(x_ref, w_ref, o_ref):
    o_ref[...] = lax.dot_general(
        x_ref[...], w_ref[...], (((1,), (0,)), ((), ())),
        precision=lax.Precision.HIGHEST, preferred_element_type=jnp.float32)


def _comb_body(z_ref, aggp_ref, degp_ref, w_ref, b_ref, eps_ref, o_ref):
    agg = aggp_ref[0, :_N, :] + aggp_ref[1, :_N, :]
    deg = jnp.maximum(degp_ref[0, :_N, 0:1] + degp_ref[1, :_N, 0:1], 1.0)
    h = (1.0 + eps_ref[...]) * z_ref[...] + agg / deg + b_ref[...]
    h = jnp.maximum(h, 0.0)
    o_ref[...] = lax.dot_general(
        h, w_ref[...], (((1,), (0,)), ((), ())),
        precision=lax.Precision.HIGHEST, preferred_element_type=jnp.float32)


def _final_body(z_ref, aggp_ref, degp_ref, b_ref, eps_ref, o_ref):
    agg = aggp_ref[0, :_N, :] + aggp_ref[1, :_N, :]
    deg = jnp.maximum(degp_ref[0, :_N, 0:1] + degp_ref[1, :_N, 0:1], 1.0)
    h = (1.0 + eps_ref[...]) * z_ref[...] + agg / deg + b_ref[...]
    rows = lax.broadcasted_iota(jnp.int32, (_N, _D), 0)
    o_ref[...] = jnp.where(rows == 0, 0.0, h)


def _sc_deg(dstv, E):
    ET = E // _NW
    CH = ET // _K
    NT = _NP // _NS
    mesh = plsc.VectorSubcoreMesh(core_axis_name="c", subcore_axis_name="s")

    scratch = [
        pltpu.VMEM((_K,), jnp.int32),
        pltpu.VMEM((_K, _D), jnp.float32),
        pltpu.VMEM_SHARED((_NP, _D), jnp.float32),
    ]

    def body(dst_hbm, deg_hbm, didx, db, deg_sh):
        cid = lax.axis_index("c")
        sid = lax.axis_index("s")
        wid = sid * _NC + cid
        nbase = sid * NT
        ebase = wid * ET

        @pl.loop(0, _K)
        def _z(r):
            for j in range(_D // _L):
                db[r, pl.ds(j * _L, _L)] = jnp.zeros((_L,), jnp.float32)

        for j in range(NT // _K):
            pltpu.sync_copy(db, deg_sh.at[pl.ds(nbase + j * _K, _K), :])

        @pl.loop(0, _K)
        def _o(r):
            for j in range(_D // _L):
                db[r, pl.ds(j * _L, _L)] = jnp.ones((_L,), jnp.float32)

        plsc.subcore_barrier()

        @pl.loop(0, CH)
        def _chunk(c):
            pltpu.sync_copy(dst_hbm.at[pl.ds(ebase + c * _K, _K)], didx)
            pltpu.sync_copy(db, deg_sh.at[didx], add=True)

        plsc.subcore_barrier()
        pltpu.sync_copy(deg_sh.at[pl.ds(nbase, NT), :],
                        deg_hbm.at[cid, pl.ds(nbase, NT), :])

    cp = pltpu.CompilerParams()
    if "needs_layout_passes" in pltpu.CompilerParams.__dataclass_fields__:
        cp = dataclasses.replace(cp, needs_layout_passes=False)
    kfn = pl.kernel(body,
                    out_type=jax.ShapeDtypeStruct((_NC, _NP, _D), jnp.float32),
                    mesh=mesh, scratch_types=scratch, compiler_params=cp)
    return kfn(dstv)


def _sc_agg(z, srcv, dstv, ewf):
    E = ewf.shape[0]
    ET = E // _NW
    CH = ET // _K
    NT = _NP // _NS
    mesh = plsc.VectorSubcoreMesh(core_axis_name="c", subcore_axis_name="s")

    out_types = jax.ShapeDtypeStruct((_NC, _NP, _D), jnp.float32)
    scratch = [
        pltpu.VMEM((_K,), jnp.int32),
        pltpu.VMEM((_K,), jnp.int32),
        pltpu.VMEM((ET,), jnp.float32),
        pltpu.VMEM((_K, _D), jnp.float32),
        pltpu.VMEM_SHARED((_NP, _D), jnp.float32),
    ]

    def body(z_hbm, src_hbm, dst_hbm, ew_hbm, agg_hbm,
             sidx, didx, ewv, rows, acc_sh):
        cid = lax.axis_index("c")
        sid = lax.axis_index("s")
        wid = sid * _NC + cid
        nbase = sid * NT
        ebase = wid * ET

        pltpu.sync_copy(ew_hbm.at[pl.ds(ebase, ET)], ewv)

        @pl.loop(0, _K)
        def _zrows(r):
            for j in range(_D // _L):
                rows[r, pl.ds(j * _L, _L)] = jnp.zeros((_L,), jnp.float32)

        for j in range(NT // _K):
            pltpu.sync_copy(rows, acc_sh.at[pl.ds(nbase + j * _K, _K), :])

        plsc.subcore_barrier()

        @pl.loop(0, CH)
        def _chunk(c):
            off = ebase + c * _K
            pltpu.sync_copy(src_hbm.at[pl.ds(off, _K)], sidx)
            pltpu.sync_copy(dst_hbm.at[pl.ds(off, _K)], didx)
            pltpu.sync_copy(z_hbm.at[sidx], rows)

            @pl.loop(0, _K)
            def _row(r):
                w = plsc.load_gather(
                    ewv, [jnp.full((_L,), c * _K + r, jnp.int32)])
                for j in range(_D // _L):
                    sl = pl.ds(j * _L, _L)
                    rows[r, sl] = rows[r, sl] * w

            pltpu.sync_copy(rows, acc_sh.at[didx], add=True)

        plsc.subcore_barrier()

        pltpu.sync_copy(acc_sh.at[pl.ds(nbase, NT), :],
                        agg_hbm.at[cid, pl.ds(nbase, NT), :])

    cp = pltpu.CompilerParams()
    if "needs_layout_passes" in pltpu.CompilerParams.__dataclass_fields__:
        cp = dataclasses.replace(cp, needs_layout_passes=False)
    kfn = pl.kernel(body, out_type=out_types, mesh=mesh,
                    scratch_types=scratch, compiler_params=cp)
    return kfn(z, srcv, dstv, ewf)


def kernel(ndata, edge_weight, W1, b1, W2, b2, eps1, eps2, eps3, edge_index):
    srcv = edge_index[0]
    dstv = edge_index[1]
    b1r = b1.reshape(1, _D)
    b2r = b2.reshape(1, _D)
    e1 = eps1.reshape(1, 1)
    e2 = eps2.reshape(1, 1)
    e3 = eps3.reshape(1, 1)
    zshape = jax.ShapeDtypeStruct((_N, _D), jnp.float32)

    z1 = pl.pallas_call(_mm_body, out_shape=zshape)(ndata, W1)
    degp = _sc_deg(dstv, edge_weight.shape[0])
    agg1 = _sc_agg(z1, srcv, dstv, edge_weight)
    z2 = pl.pallas_call(_comb_body, out_shape=zshape)(
        z1, agg1, degp, W2, b1r, e1)
    agg2 = _sc_agg(z2, srcv, dstv, edge_weight)
    z3 = pl.pallas_call(_comb_body, out_shape=zshape)(
        z2, agg2, degp, W2, b2r, e2)
    agg3 = _sc_agg(z3, srcv, dstv, edge_weight)
    out = pl.pallas_call(_final_body, out_shape=zshape)(
        z3, agg3, degp, b2r, e3)
    return out


# speedup vs baseline: 3.9340x; 1.0630x over previous
import dataclasses
import functools

import jax
import jax.numpy as jnp
from jax import lax
from jax.experimental import pallas as pl
from jax.experimental.pallas import tpu as pltpu
from jax.experimental.pallas import tpu_sc as plsc

_N = 10000
_D = 128
_NP = 10240
_K = 40
_NC = 2
_NS = 16
_NW = _NC * _NS
_L = 16


def _sc_compiler_params():
    cp = pltpu.CompilerParams()
    fields = pltpu.CompilerParams.__dataclass_fields__
    if "needs_layout_passes" in fields:
        cp = dataclasses.replace(cp, needs_layout_passes=False)
    if "use_tc_tiling_on_sc" in fields:
        cp = dataclasses.replace(cp, use_tc_tiling_on_sc=False)
    return cp


def _mm_body(x_ref, w_ref, o_ref):
    o_ref[...] = lax.dot_general(
        x_ref[...], w_ref[...], (((1,), (0,)), ((), ())),
        precision=lax.Precision.HIGHEST, preferred_element_type=jnp.float32)


def _comb_body(z_ref, aggp_ref, degp_ref, w_ref, b_ref, eps_ref, o_ref):
    agg = aggp_ref[0, :_N, :] + aggp_ref[1, :_N, :]
    deg = jnp.maximum(degp_ref[0, :_N, 0:1] + degp_ref[1, :_N, 0:1], 1.0)
    h = (1.0 + eps_ref[...]) * z_ref[...] + agg / deg + b_ref[...]
    h = jnp.maximum(h, 0.0)
    o_ref[...] = lax.dot_general(
        h, w_ref[...], (((1,), (0,)), ((), ())),
        precision=lax.Precision.HIGHEST, preferred_element_type=jnp.float32)


def _final_body(z_ref, aggp_ref, degp_ref, b_ref, eps_ref, o_ref):
    agg = aggp_ref[0, :_N, :] + aggp_ref[1, :_N, :]
    deg = jnp.maximum(degp_ref[0, :_N, 0:1] + degp_ref[1, :_N, 0:1], 1.0)
    h = (1.0 + eps_ref[...]) * z_ref[...] + agg / deg + b_ref[...]
    rows = lax.broadcasted_iota(jnp.int32, (_N, _D), 0)
    o_ref[...] = jnp.where(rows == 0, 0.0, h)


def _zero_vmem_rows(buf, nrows):
    @pl.loop(0, nrows)
    def _z(r):
        for j in range(_D // _L):
            buf[r, pl.ds(j * _L, _L)] = jnp.zeros((_L,), jnp.float32)


def _sc_deg(dstm, E):
    ET = E // _NW
    CH = ET // _K
    NT = _NP // _NS
    FIRE = 5
    mesh = plsc.VectorSubcoreMesh(core_axis_name="c", subcore_axis_name="s")

    scratch = [
        pltpu.VMEM((CH, _K), jnp.int32),
        pltpu.VMEM((_K, _D), jnp.float32),
        pltpu.VMEM_SHARED((_NP, _D), jnp.float32),
        pltpu.SemaphoreType.DMA,
    ]

    def body(dst_hbm, deg_hbm, didx, db, deg_sh, sem):
        cid = lax.axis_index("c")
        sid = lax.axis_index("s")
        wid = sid * _NC + cid
        nbase = sid * NT

        pltpu.sync_copy(dst_hbm.at[wid], didx)
        _zero_vmem_rows(db, _K)
        for j in range(NT // _K):
            pltpu.sync_copy(db, deg_sh.at[pl.ds(nbase + j * _K, _K), :])

        @pl.loop(0, _K)
        def _o(r):
            for j in range(_D // _L):
                db[r, pl.ds(j * _L, _L)] = jnp.ones((_L,), jnp.float32)

        plsc.subcore_barrier()

        @pl.loop(0, CH // FIRE)
        def _grp(g):
            for t in range(FIRE):
                pltpu.async_copy(db, deg_sh.at[didx.at[g * FIRE + t]], sem,
                                 add=True)
            for t in range(FIRE):
                pltpu.make_async_copy(
                    db, deg_sh.at[didx.at[g * FIRE + t]], sem).wait()

        plsc.subcore_barrier()
        pltpu.sync_copy(deg_sh.at[pl.ds(nbase, NT), :],
                        deg_hbm.at[cid, pl.ds(nbase, NT), :])

    kfn = pl.kernel(body,
                    out_type=jax.ShapeDtypeStruct((_NC, _NP, _D), jnp.float32),
                    mesh=mesh, scratch_types=scratch,
                    compiler_params=_sc_compiler_params())
    return kfn(dstm)


def _sc_agg(z, srcm, dstm, ewf):
    E = ewf.shape[0]
    ET = E // _NW
    CH = ET // _K
    NT = _NP // _NS
    mesh = plsc.VectorSubcoreMesh(core_axis_name="c", subcore_axis_name="s")

    scratch = [
        pltpu.VMEM((CH, _K), jnp.int32),
        pltpu.VMEM((CH, _K), jnp.int32),
        pltpu.VMEM((_K,), jnp.float32),
        pltpu.VMEM((_K,), jnp.float32),
        pltpu.VMEM((_K, _D), jnp.float32),
        pltpu.VMEM((_K, _D), jnp.float32),
        pltpu.VMEM((_K, _D), jnp.float32),
        pltpu.VMEM((_K, _D), jnp.float32),
        pltpu.VMEM_SHARED((_NP, _D), jnp.float32),
        pltpu.SemaphoreType.DMA,
        pltpu.SemaphoreType.DMA,
        pltpu.SemaphoreType.DMA,
        pltpu.SemaphoreType.DMA,
    ]

    def body(z_hbm, src_hbm, dst_hbm, ew_hbm, agg_hbm,
             sidx, didx, ew0, ew1, gb0, gb1, sb0, sb1, acc_sh,
             gs0, gs1, ss0, ss1):
        gb = (gb0, gb1)
        sb = (sb0, sb1)
        ew = (ew0, ew1)
        gs = (gs0, gs1)
        ss = (ss0, ss1)
        cid = lax.axis_index("c")
        sid = lax.axis_index("s")
        wid = sid * _NC + cid
        nbase = sid * NT

        pltpu.sync_copy(src_hbm.at[wid], sidx)
        pltpu.sync_copy(dst_hbm.at[wid], didx)

        _zero_vmem_rows(sb0, _K)
        for j in range(NT // _K):
            pltpu.sync_copy(sb0, acc_sh.at[pl.ds(nbase + j * _K, _K), :])
        plsc.subcore_barrier()

        ebase = wid * ET
        pltpu.async_copy(z_hbm.at[sidx.at[0]], gb0, gs0)
        pltpu.async_copy(ew_hbm.at[pl.ds(ebase, _K)], ew0, gs0)
        pltpu.async_copy(z_hbm.at[sidx.at[1]], gb1, gs1)
        pltpu.async_copy(ew_hbm.at[pl.ds(ebase + _K, _K)], ew1, gs1)

        @pl.loop(0, CH // 2)
        def _pair(p):
            for b in (0, 1):
                c = 2 * p + b
                pltpu.make_async_copy(z_hbm.at[sidx.at[c]], gb[b],
                                      gs[b]).wait()
                pltpu.make_async_copy(ew_hbm.at[pl.ds(ebase + c * _K, _K)],
                                      ew[b], gs[b]).wait()

                @pl.when(c >= 2)
                def _ws():
                    pltpu.make_async_copy(sb[b], acc_sh.at[didx.at[c]],
                                          ss[b]).wait()

                @pl.loop(0, _K)
                def _row(r):
                    w = plsc.load_gather(
                        ew[b], [jnp.full((_L,), r, jnp.int32)])
                    for j in range(_D // _L):
                        sl = pl.ds(j * _L, _L)
                        sb[b][r, sl] = gb[b][r, sl] * w

                @pl.when(c + 2 < CH)
                def _ng():
                    pltpu.async_copy(z_hbm.at[sidx.at[c + 2]], gb[b], gs[b])
                    pltpu.async_copy(
                        ew_hbm.at[pl.ds(ebase + (c + 2) * _K, _K)],
                        ew[b], gs[b])

                pltpu.async_copy(sb[b], acc_sh.at[didx.at[c]], ss[b],
                                 add=True)

        pltpu.make_async_copy(sb0, acc_sh.at[didx.at[CH - 2]], ss0).wait()
        pltpu.make_async_copy(sb1, acc_sh.at[didx.at[CH - 1]], ss1).wait()
        plsc.subcore_barrier()

        pltpu.sync_copy(acc_sh.at[pl.ds(nbase, NT), :],
                        agg_hbm.at[cid, pl.ds(nbase, NT), :])

    kfn = pl.kernel(body,
                    out_type=jax.ShapeDtypeStruct((_NC, _NP, _D), jnp.float32),
                    mesh=mesh, scratch_types=scratch,
                    compiler_params=_sc_compiler_params())
    return kfn(z, srcm, dstm, ewf)


def kernel(ndata, edge_weight, W1, b1, W2, b2, eps1, eps2, eps3, edge_index):
    E = edge_weight.shape[0]
    CH = E // _NW // _K
    srcm = edge_index[0].reshape(_NW, CH, _K)
    dstm = edge_index[1].reshape(_NW, CH, _K)
    b1r = b1.reshape(1, _D)
    b2r = b2.reshape(1, _D)
    e1 = eps1.reshape(1, 1)
    e2 = eps2.reshape(1, 1)
    e3 = eps3.reshape(1, 1)
    zshape = jax.ShapeDtypeStruct((_N, _D), jnp.float32)

    z1 = pl.pallas_call(_mm_body, out_shape=zshape)(ndata, W1)
    degp = _sc_deg(dstm, E)
    agg1 = _sc_agg(z1, srcm, dstm, edge_weight)
    z2 = pl.pallas_call(_comb_body, out_shape=zshape)(
        z1, agg1, degp, W2, b1r, e1)
    agg2 = _sc_agg(z2, srcm, dstm, edge_weight)
    z3 = pl.pallas_call(_comb_body, out_shape=zshape)(
        z2, agg2, degp, W2, b2r, e2)
    agg3 = _sc_agg(z3, srcm, dstm, edge_weight)
    out = pl.pallas_call(_final_body, out_shape=zshape)(
        z3, agg3, degp, b2r, e3)
    return out


# speedup vs baseline: 3.9548x; 1.0053x over previous
import dataclasses
import functools

import jax
import jax.numpy as jnp
from jax import lax
from jax.experimental import pallas as pl
from jax.experimental.pallas import tpu as pltpu
from jax.experimental.pallas import tpu_sc as plsc

_N = 10000
_D = 128
_NP = 10240
_K = 40
_NC = 2
_NS = 16
_NW = _NC * _NS
_L = 16


def _sc_compiler_params():
    cp = pltpu.CompilerParams()
    fields = pltpu.CompilerParams.__dataclass_fields__
    if "needs_layout_passes" in fields:
        cp = dataclasses.replace(cp, needs_layout_passes=False)
    if "use_tc_tiling_on_sc" in fields:
        cp = dataclasses.replace(cp, use_tc_tiling_on_sc=False)
    return cp


def _mm_body(x_ref, w_ref, o_ref):
    o_ref[...] = lax.dot_general(
        x_ref[...], w_ref[...], (((1,), (0,)), ((), ())),
        precision=lax.Precision.HIGHEST, preferred_element_type=jnp.float32)


def _comb_body(z_ref, aggp_ref, degp_ref, w_ref, b_ref, eps_ref, o_ref):
    agg = aggp_ref[0, :_N, :] + aggp_ref[1, :_N, :]
    deg = jnp.maximum(degp_ref[0, :_N, 0:1] + degp_ref[1, :_N, 0:1], 1.0)
    h = (1.0 + eps_ref[...]) * z_ref[...] + agg / deg + b_ref[...]
    h = jnp.maximum(h, 0.0)
    o_ref[...] = lax.dot_general(
        h, w_ref[...], (((1,), (0,)), ((), ())),
        precision=lax.Precision.HIGHEST, preferred_element_type=jnp.float32)


def _final_body(z_ref, aggp_ref, degp_ref, b_ref, eps_ref, o_ref):
    agg = aggp_ref[0, :_N, :] + aggp_ref[1, :_N, :]
    deg = jnp.maximum(degp_ref[0, :_N, 0:1] + degp_ref[1, :_N, 0:1], 1.0)
    h = (1.0 + eps_ref[...]) * z_ref[...] + agg / deg + b_ref[...]
    rows = lax.broadcasted_iota(jnp.int32, (_N, _D), 0)
    o_ref[...] = jnp.where(rows == 0, 0.0, h)


def _zero_vmem_rows(buf, nrows):
    @pl.loop(0, nrows)
    def _z(r):
        for j in range(_D // _L):
            buf[r, pl.ds(j * _L, _L)] = jnp.zeros((_L,), jnp.float32)


def _sc_deg(dstm, E):
    ET = E // _NW
    CH = ET // _K
    NT = _NP // _NS
    FIRE = 5
    mesh = plsc.VectorSubcoreMesh(core_axis_name="c", subcore_axis_name="s")

    scratch = [
        pltpu.VMEM((CH, _K), jnp.int32),
        pltpu.VMEM((_K, _D), jnp.float32),
        pltpu.VMEM_SHARED((_NP, _D), jnp.float32),
        pltpu.SemaphoreType.DMA,
    ]

    def body(dst_hbm, deg_hbm, didx, db, deg_sh, sem):
        cid = lax.axis_index("c")
        sid = lax.axis_index("s")
        wid = sid * _NC + cid
        nbase = sid * NT

        pltpu.sync_copy(dst_hbm.at[wid], didx)
        _zero_vmem_rows(db, _K)
        for j in range(NT // _K):
            pltpu.sync_copy(db, deg_sh.at[pl.ds(nbase + j * _K, _K), :])

        @pl.loop(0, _K)
        def _o(r):
            for j in range(_D // _L):
                db[r, pl.ds(j * _L, _L)] = jnp.ones((_L,), jnp.float32)

        plsc.subcore_barrier()

        @pl.loop(0, CH // FIRE)
        def _grp(g):
            for t in range(FIRE):
                pltpu.async_copy(db, deg_sh.at[didx.at[g * FIRE + t]], sem,
                                 add=True)
            for t in range(FIRE):
                pltpu.make_async_copy(
                    db, deg_sh.at[didx.at[g * FIRE + t]], sem).wait()

        plsc.subcore_barrier()
        pltpu.sync_copy(deg_sh.at[pl.ds(nbase, NT), :],
                        deg_hbm.at[cid, pl.ds(nbase, NT), :])

    kfn = pl.kernel(body,
                    out_type=jax.ShapeDtypeStruct((_NC, _NP, _D), jnp.float32),
                    mesh=mesh, scratch_types=scratch,
                    compiler_params=_sc_compiler_params())
    return kfn(dstm)


def _sc_agg(z, srcm, dstm, ewf):
    E = ewf.shape[0]
    ET = E // _NW
    CH = ET // _K
    NT = _NP // _NS
    mesh = plsc.VectorSubcoreMesh(core_axis_name="c", subcore_axis_name="s")

    scratch = [
        pltpu.VMEM((CH, _K), jnp.int32),
        pltpu.VMEM((CH, _K), jnp.int32),
        pltpu.VMEM((_K,), jnp.float32),
        pltpu.VMEM((_K,), jnp.float32),
        pltpu.VMEM((_K, _D), jnp.float32),
        pltpu.VMEM((_K, _D), jnp.float32),
        pltpu.VMEM((_K, _D), jnp.float32),
        pltpu.VMEM((_K, _D), jnp.float32),
        pltpu.VMEM_SHARED((_NP, _D), jnp.float32),
        pltpu.SemaphoreType.DMA,
        pltpu.SemaphoreType.DMA,
        pltpu.SemaphoreType.DMA,
        pltpu.SemaphoreType.DMA,
    ]

    def body(z_hbm, src_hbm, dst_hbm, ew_hbm, agg_hbm,
             sidx, didx, ew0, ew1, gb0, gb1, sb0, sb1, acc_sh,
             gs0, gs1, ss0, ss1):
        gb = (gb0, gb1)
        sb = (sb0, sb1)
        ew = (ew0, ew1)
        gs = (gs0, gs1)
        ss = (ss0, ss1)
        cid = lax.axis_index("c")
        sid = lax.axis_index("s")
        wid = sid * _NC + cid
        nbase = sid * NT

        pltpu.sync_copy(src_hbm.at[wid], sidx)
        pltpu.sync_copy(dst_hbm.at[wid], didx)

        _zero_vmem_rows(sb0, _K)
        for j in range(NT // _K):
            pltpu.sync_copy(sb0, acc_sh.at[pl.ds(nbase + j * _K, _K), :])
        plsc.subcore_barrier()

        ebase = wid * ET
        pltpu.async_copy(z_hbm.at[sidx.at[0]], gb0, gs0)
        pltpu.async_copy(ew_hbm.at[pl.ds(ebase, _K)], ew0, gs0)
        pltpu.async_copy(z_hbm.at[sidx.at[1]], gb1, gs1)
        pltpu.async_copy(ew_hbm.at[pl.ds(ebase + _K, _K)], ew1, gs1)

        @pl.loop(0, CH // 2)
        def _pair(p):
            for b in (0, 1):
                c = 2 * p + b
                pltpu.make_async_copy(z_hbm.at[sidx.at[c]], gb[b],
                                      gs[b]).wait()
                pltpu.make_async_copy(ew_hbm.at[pl.ds(ebase + c * _K, _K)],
                                      ew[b], gs[b]).wait()

                @pl.when(c >= 2)
                def _ws():
                    pltpu.make_async_copy(sb[b], acc_sh.at[didx.at[c]],
                                          ss[b]).wait()

                @pl.loop(0, _K // 8)
                def _rows(g):
                    for i in range(8):
                        r = g * 8 + i
                        w = plsc.load_gather(
                            ew[b], [jnp.full((_L,), r, jnp.int32)])
                        for j in range(_D // _L):
                            sl = pl.ds(j * _L, _L)
                            sb[b][r, sl] = gb[b][r, sl] * w

                @pl.when(c + 2 < CH)
                def _ng():
                    pltpu.async_copy(z_hbm.at[sidx.at[c + 2]], gb[b], gs[b])
                    pltpu.async_copy(
                        ew_hbm.at[pl.ds(ebase + (c + 2) * _K, _K)],
                        ew[b], gs[b])

                pltpu.async_copy(sb[b], acc_sh.at[didx.at[c]], ss[b],
                                 add=True)

        pltpu.make_async_copy(sb0, acc_sh.at[didx.at[CH - 2]], ss0).wait()
        pltpu.make_async_copy(sb1, acc_sh.at[didx.at[CH - 1]], ss1).wait()
        plsc.subcore_barrier()

        pltpu.sync_copy(acc_sh.at[pl.ds(nbase, NT), :],
                        agg_hbm.at[cid, pl.ds(nbase, NT), :])

    kfn = pl.kernel(body,
                    out_type=jax.ShapeDtypeStruct((_NC, _NP, _D), jnp.float32),
                    mesh=mesh, scratch_types=scratch,
                    compiler_params=_sc_compiler_params())
    return kfn(z, srcm, dstm, ewf)


def kernel(ndata, edge_weight, W1, b1, W2, b2, eps1, eps2, eps3, edge_index):
    E = edge_weight.shape[0]
    CH = E // _NW // _K
    srcm = edge_index[0].reshape(_NW, CH, _K)
    dstm = edge_index[1].reshape(_NW, CH, _K)
    b1r = b1.reshape(1, _D)
    b2r = b2.reshape(1, _D)
    e1 = eps1.reshape(1, 1)
    e2 = eps2.reshape(1, 1)
    e3 = eps3.reshape(1, 1)
    zshape = jax.ShapeDtypeStruct((_N, _D), jnp.float32)

    z1 = pl.pallas_call(_mm_body, out_shape=zshape)(ndata, W1)
    degp = _sc_deg(dstm, E)
    agg1 = _sc_agg(z1, srcm, dstm, edge_weight)
    z2 = pl.pallas_call(_comb_body, out_shape=zshape)(
        z1, agg1, degp, W2, b1r, e1)
    agg2 = _sc_agg(z2, srcm, dstm, edge_weight)
    z3 = pl.pallas_call(_comb_body, out_shape=zshape)(
        z2, agg2, degp, W2, b2r, e2)
    agg3 = _sc_agg(z3, srcm, dstm, edge_weight)
    out = pl.pallas_call(_final_body, out_shape=zshape)(
        z3, agg3, degp, b2r, e3)
    return out
